# baseline (speedup 1.0000x reference)
import functools

import jax
import jax.numpy as jnp
from jax import lax
from jax.experimental import pallas as pl
from jax.experimental.pallas import tpu as pltpu
from jax.experimental.pallas import tpu_sc as plsc

_NC = 2
_NS = 16
_NW = _NC * _NS


def _make_sc_gather(B, D, C, nbuf=3):
    b_per_w = B // _NW
    n_chunks = b_per_w // C
    assert B % (_NW * C) == 0
    n_main = n_chunks - (nbuf - 1)
    assert n_main % nbuf == 0
    Dh = D // 2
    mesh = plsc.VectorSubcoreMesh(core_axis_name="c", subcore_axis_name="s")

    @functools.partial(
        pl.kernel,
        mesh=mesh,
        out_type=jax.ShapeDtypeStruct((B, D), jnp.float32),
        scratch_types=[
            pltpu.VMEM((b_per_w,), jnp.int32),
            pltpu.VMEM((nbuf, C, D), jnp.float32),
        ]
        + [pltpu.SemaphoreType.DMA] * (3 * nbuf),
    )
    def gather_rows(idx_hbm, table_hbm, out_hbm, idx_v, rows_v, *sems):
        sem_a, sem_b, sem_w = sems[:nbuf], sems[nbuf : 2 * nbuf], sems[2 * nbuf :]
        wid = lax.axis_index("s") * _NC + lax.axis_index("c")
        base = wid * b_per_w
        pltpu.sync_copy(idx_hbm.at[pl.ds(base, b_per_w)], idx_v)

        def gather_start(i, b):
            idx = idx_v.at[pl.ds(i * C, C)]
            pltpu.make_async_copy(
                table_hbm.at[idx, pl.ds(0, Dh)],
                rows_v.at[b, :, pl.ds(0, Dh)],
                sem_a[b],
            ).start()
            pltpu.make_async_copy(
                table_hbm.at[idx, pl.ds(Dh, Dh)],
                rows_v.at[b, :, pl.ds(Dh, Dh)],
                sem_b[b],
            ).start()

        def gather_wait(i, b):
            idx = idx_v.at[pl.ds(i * C, C)]
            pltpu.make_async_copy(
                table_hbm.at[idx, pl.ds(0, Dh)],
                rows_v.at[b, :, pl.ds(0, Dh)],
                sem_a[b],
            ).wait()
            pltpu.make_async_copy(
                table_hbm.at[idx, pl.ds(Dh, Dh)],
                rows_v.at[b, :, pl.ds(Dh, Dh)],
                sem_b[b],
            ).wait()

        def write(i, b):
            return pltpu.make_async_copy(
                rows_v.at[b], out_hbm.at[pl.ds(base + i * C, C)], sem_w[b]
            )

        for b in range(nbuf - 1):
            gather_start(b, b)

        def step(i, b, b2):
            gather_wait(i, b)
            write(i, b).start()

            @pl.when(i >= 1)
            def _():
                write(i - 1, b2).wait()

            gather_start(i + nbuf - 1, b2)

        def round_body(g, carry):
            i0 = g * nbuf
            for r in range(nbuf):
                step(i0 + r, r, (r + nbuf - 1) % nbuf)
            return carry

        lax.fori_loop(0, n_main // nbuf, round_body, 0)

        for i in range(n_main, n_chunks):
            b = i % nbuf
            gather_wait(i, b)
            write(i, b).start()
            write(i - 1, (i - 1) % nbuf).wait()
        write(n_chunks - 1, (n_chunks - 1) % nbuf).wait()

    return gather_rows


def kernel(position_ids, pos_enc):
    b, s = position_ids.shape
    _, d = pos_enc.shape
    idx = position_ids.reshape(b * s).astype(jnp.int32)
    out = _make_sc_gather(b * s, d, 8, nbuf=3)(idx, pos_enc)
    return out.reshape(b, s, d)
